# Initial kernel scaffold; baseline (speedup 1.0000x reference)
import functools

import jax
import jax.numpy as jnp
from jax import lax
from jax.experimental import pallas as pl
from jax.experimental.pallas import tpu as pltpu
from jax.experimental.pallas import tpu_sc as plsc

N = 10000
F = 128
E = 320000
NC = 2
NS = 16
NW = NC * NS
CH = 128
CHUNKS = 79
EPT = CH * CHUNKS
E_PAD = EPT * NW
NPAD = 10240
NPT = NPAD // NS


_sc_mesh = plsc.VectorSubcoreMesh(core_axis_name="c", subcore_axis_name="s")


def _deg_body(dst_flat, deg_out, idx_v, hist_v):
    c = lax.axis_index("c")
    s = lax.axis_index("s")
    wid = s * NC + c

    zeros16 = jnp.zeros((16,), jnp.float32)

    def zero_body(i, carry):
        hist_v[pl.ds(i * 16, 16)] = zeros16
        return carry

    lax.fori_loop(0, NPAD // 16, zero_body, 0)

    pltpu.sync_copy(dst_flat.at[pl.ds(wid * EPT, EPT)], idx_v)

    ones16 = jnp.ones((16,), jnp.float32)

    def acc_body(j, carry):
        dv = idx_v[pl.ds(j * 16, 16)]
        plsc.addupdate_scatter(hist_v, [dv], ones16)
        return carry

    lax.fori_loop(0, EPT // 16, acc_body, 0)

    pltpu.sync_copy(hist_v, deg_out.at[wid])


_deg_kernel = functools.partial(
    pl.kernel,
    out_type=jax.ShapeDtypeStruct((NW, NPAD), jnp.float32),
    mesh=_sc_mesh,
    scratch_types=[
        pltpu.VMEM((EPT,), jnp.int32),
        pltpu.VMEM((NPAD,), jnp.float32),
    ],
)(_deg_body)


def _scale_body(deg_ref, x_ref, y_ref, dinv_ref):
    degsum = jnp.sum(deg_ref[...], axis=0, keepdims=True) + 1.0
    dinv_row = lax.rsqrt(degsum)
    dinv_col = dinv_row.reshape(NPAD, 1)
    dinv_ref[...] = dinv_col
    y_ref[...] = x_ref[...] * dinv_col


def _scale_call(deg_p, x_ext):
    return pl.pallas_call(
        _scale_body,
        out_shape=(
            jax.ShapeDtypeStruct((NPAD, F), jnp.float32),
            jax.ShapeDtypeStruct((NPAD, 1), jnp.float32),
        ),
    )(deg_p, x_ext)


def _prop_body(src3, dst3, y_hbm, zeros_blk, zp_out,
               src_v, dst_v, rows_v, z_sp, sem):
    c = lax.axis_index("c")
    s = lax.axis_index("s")
    wid = s * NC + c

    pltpu.sync_copy(zeros_blk, rows_v)
    for i in range(NPT // CH):
        pltpu.sync_copy(rows_v, z_sp.at[pl.ds(s * NPT + i * CH, CH)])

    pltpu.sync_copy(src3.at[wid], src_v)
    pltpu.sync_copy(dst3.at[wid], dst_v)

    plsc.subcore_barrier()

    def chunk_body(j, carry):
        pltpu.async_copy(y_hbm.at[src_v.at[j]], rows_v, sem).wait()
        pltpu.sync_copy(rows_v, z_sp.at[dst_v.at[j]], add=True)
        return carry

    lax.fori_loop(0, CHUNKS, chunk_body, 0)

    plsc.subcore_barrier()

    for i in range(NPT // CH):
        off = s * NPT + i * CH
        pltpu.sync_copy(z_sp.at[pl.ds(off, CH)], rows_v)
        pltpu.sync_copy(rows_v, zp_out.at[c, pl.ds(off, CH)])


_prop_kernel = functools.partial(
    pl.kernel,
    out_type=jax.ShapeDtypeStruct((NC, NPAD, F), jnp.float32),
    mesh=_sc_mesh,
    scratch_types=[
        pltpu.VMEM((CHUNKS, CH), jnp.int32),
        pltpu.VMEM((CHUNKS, CH), jnp.int32),
        pltpu.VMEM((CH, F), jnp.float32),
        pltpu.VMEM_SHARED((NPAD, F), jnp.float32),
        pltpu.SemaphoreType.DMA,
    ],
)(_prop_body)


_DBLK = 2048


def _final_body(z_ref, y_ref, dinv_ref, x_ref, w_ref, b_ref, o_ref):
    z = z_ref[0] + z_ref[1] + y_ref[...]
    h = z * dinv_ref[...]
    out = lax.dot_general(h, w_ref[...], (((1,), (1,)), ((), ())),
                          preferred_element_type=jnp.float32)
    o_ref[...] = x_ref[...] + jnp.maximum(out + b_ref[...], 0.0)


def _final_call(z_p, y_ext, dinv_col, x_ext, W, b2):
    grid = NPAD // _DBLK
    return pl.pallas_call(
        _final_body,
        grid=(grid,),
        in_specs=[
            pl.BlockSpec((NC, _DBLK, F), lambda i: (0, i, 0)),
            pl.BlockSpec((_DBLK, F), lambda i: (i, 0)),
            pl.BlockSpec((_DBLK, 1), lambda i: (i, 0)),
            pl.BlockSpec((_DBLK, F), lambda i: (i, 0)),
            pl.BlockSpec((F, F), lambda i: (0, 0)),
            pl.BlockSpec((1, F), lambda i: (0, 0)),
        ],
        out_specs=pl.BlockSpec((_DBLK, F), lambda i: (i, 0)),
        out_shape=jax.ShapeDtypeStruct((N, F), jnp.float32),
    )(z_p, y_ext, dinv_col, x_ext, W, b2)


def kernel(x, edge_index, W, b):
    src = edge_index[0].astype(jnp.int32)
    dst = edge_index[1].astype(jnp.int32)
    pad = jnp.full((E_PAD - E,), N, jnp.int32)
    src3 = jnp.concatenate([src, pad]).reshape(NW, CHUNKS, CH)
    dst3 = jnp.concatenate([dst, pad]).reshape(NW, CHUNKS, CH)
    dst_flat = dst3.reshape(E_PAD)
    x_ext = jnp.concatenate(
        [x.astype(jnp.float32), jnp.zeros((NPAD - N, F), jnp.float32)])
    zeros_blk = jnp.zeros((CH, F), jnp.float32)
    b2 = b.reshape(1, F).astype(jnp.float32)

    deg_p = _deg_kernel(dst_flat)
    y_ext, dinv_col = _scale_call(deg_p, x_ext)
    z_p = _prop_kernel(src3, dst3, y_ext, zeros_blk)
    return _final_call(z_p, y_ext, dinv_col, x_ext, W.astype(jnp.float32), b2)


# speedup vs baseline: 17.3123x; 17.3123x over previous
import functools

import jax
import jax.numpy as jnp
from jax import lax
from jax.experimental import pallas as pl
from jax.experimental.pallas import tpu as pltpu
from jax.experimental.pallas import tpu_sc as plsc

N = 10000
F = 128
E = 320000
NC = 2
NS = 16
NW = NC * NS
CH = 128
CHUNKS = 79
EPT = CH * CHUNKS
E_PAD = EPT * NW
NPAD = 10240
NPT = NPAD // NS


_sc_mesh = plsc.VectorSubcoreMesh(core_axis_name="c", subcore_axis_name="s")
_sc_params = pltpu.CompilerParams(needs_layout_passes=False)


def _deg_body(dst_flat, deg_out, idx_v, hist_v):
    c = lax.axis_index("c")
    s = lax.axis_index("s")
    wid = s * NC + c

    zeros16 = jnp.zeros((16,), jnp.float32)

    def zero_body(i, carry):
        hist_v[pl.ds(i * 16, 16)] = zeros16
        return carry

    lax.fori_loop(0, NPAD // 16, zero_body, 0)

    pltpu.sync_copy(dst_flat.at[pl.ds(wid * EPT, EPT)], idx_v)

    ones16 = jnp.ones((16,), jnp.float32)

    def acc_body(j, carry):
        dv = idx_v[pl.ds(j * 16, 16)]
        plsc.addupdate_scatter(hist_v, [dv], ones16)
        return carry

    lax.fori_loop(0, EPT // 16, acc_body, 0)

    pltpu.sync_copy(hist_v, deg_out.at[wid])


_deg_kernel = functools.partial(
    pl.kernel,
    out_type=jax.ShapeDtypeStruct((NW, NPAD), jnp.float32),
    mesh=_sc_mesh,
    compiler_params=_sc_params,
    scratch_types=[
        pltpu.VMEM((EPT,), jnp.int32),
        pltpu.VMEM((NPAD,), jnp.float32),
    ],
)(_deg_body)


def _scale_body(deg_ref, x_ref, y_ref, dinv_ref):
    degsum = jnp.sum(deg_ref[...], axis=0, keepdims=True) + 1.0
    dinv_row = lax.rsqrt(degsum)
    dinv_col = dinv_row.reshape(NPAD, 1)
    dinv_ref[...] = dinv_col
    y_ref[...] = x_ref[...] * dinv_col


def _scale_call(deg_p, x_ext):
    return pl.pallas_call(
        _scale_body,
        out_shape=(
            jax.ShapeDtypeStruct((NPAD, F), jnp.float32),
            jax.ShapeDtypeStruct((NPAD, 1), jnp.float32),
        ),
    )(deg_p, x_ext)


def _prop_body(src3, dst3, y_hbm, zeros_blk, zp_out,
               src_v, dst_v, rows_v, z_sp, sem):
    c = lax.axis_index("c")
    s = lax.axis_index("s")
    wid = s * NC + c

    pltpu.sync_copy(zeros_blk, rows_v)
    for i in range(NPT // CH):
        pltpu.sync_copy(rows_v, z_sp.at[pl.ds(s * NPT + i * CH, CH)])

    pltpu.sync_copy(src3.at[wid], src_v)
    pltpu.sync_copy(dst3.at[wid], dst_v)

    plsc.subcore_barrier()

    def chunk_body(j, carry):
        pltpu.async_copy(y_hbm.at[src_v.at[j]], rows_v, sem).wait()
        pltpu.sync_copy(rows_v, z_sp.at[dst_v.at[j]], add=True)
        return carry

    lax.fori_loop(0, CHUNKS, chunk_body, 0)

    plsc.subcore_barrier()

    for i in range(NPT // CH):
        off = s * NPT + i * CH
        pltpu.sync_copy(z_sp.at[pl.ds(off, CH)], rows_v)
        pltpu.sync_copy(rows_v, zp_out.at[c, pl.ds(off, CH)])


_prop_kernel = functools.partial(
    pl.kernel,
    out_type=jax.ShapeDtypeStruct((NC, NPAD, F), jnp.float32),
    mesh=_sc_mesh,
    compiler_params=_sc_params,
    scratch_types=[
        pltpu.VMEM((CHUNKS, CH), jnp.int32),
        pltpu.VMEM((CHUNKS, CH), jnp.int32),
        pltpu.VMEM((CH, F), jnp.float32),
        pltpu.VMEM_SHARED((NPAD, F), jnp.float32),
        pltpu.SemaphoreType.DMA,
    ],
)(_prop_body)


_DBLK = 2048


def _final_body(z_ref, y_ref, dinv_ref, x_ref, w_ref, b_ref, o_ref):
    z = z_ref[0] + z_ref[1] + y_ref[...]
    h = z * dinv_ref[...]
    out = lax.dot_general(h, w_ref[...], (((1,), (1,)), ((), ())),
                          preferred_element_type=jnp.float32)
    o_ref[...] = x_ref[...] + jnp.maximum(out + b_ref[...], 0.0)


def _final_call(z_p, y_ext, dinv_col, x_ext, W, b2):
    grid = NPAD // _DBLK
    return pl.pallas_call(
        _final_body,
        grid=(grid,),
        in_specs=[
            pl.BlockSpec((NC, _DBLK, F), lambda i: (0, i, 0)),
            pl.BlockSpec((_DBLK, F), lambda i: (i, 0)),
            pl.BlockSpec((_DBLK, 1), lambda i: (i, 0)),
            pl.BlockSpec((_DBLK, F), lambda i: (i, 0)),
            pl.BlockSpec((F, F), lambda i: (0, 0)),
            pl.BlockSpec((1, F), lambda i: (0, 0)),
        ],
        out_specs=pl.BlockSpec((_DBLK, F), lambda i: (i, 0)),
        out_shape=jax.ShapeDtypeStruct((N, F), jnp.float32),
    )(z_p, y_ext, dinv_col, x_ext, W, b2)


def kernel(x, edge_index, W, b):
    src = edge_index[0].astype(jnp.int32)
    dst = edge_index[1].astype(jnp.int32)
    pad = jnp.full((E_PAD - E,), N, jnp.int32)
    src3 = jnp.concatenate([src, pad]).reshape(NW, CHUNKS, CH)
    dst3 = jnp.concatenate([dst, pad]).reshape(NW, CHUNKS, CH)
    dst_flat = dst3.reshape(E_PAD)
    x_ext = jnp.concatenate(
        [x.astype(jnp.float32), jnp.zeros((NPAD - N, F), jnp.float32)])
    zeros_blk = jnp.zeros((CH, F), jnp.float32)
    b2 = b.reshape(1, F).astype(jnp.float32)

    deg_p = _deg_kernel(dst_flat)
    y_ext, dinv_col = _scale_call(deg_p, x_ext)
    z_p = _prop_kernel(src3, dst3, y_ext, zeros_blk)
    return _final_call(z_p, y_ext, dinv_col, x_ext, W.astype(jnp.float32), b2)


# speedup vs baseline: 33.7533x; 1.9497x over previous
import functools

import jax
import jax.numpy as jnp
from jax import lax
from jax.experimental import pallas as pl
from jax.experimental.pallas import tpu as pltpu
from jax.experimental.pallas import tpu_sc as plsc

N = 10000
F = 128
E = 320000
NC = 2
NS = 16
NW = NC * NS
CH = 128
CHUNKS = 80
EPT = CH * CHUNKS
E_PAD = EPT * NW
NPAD = 10240
NPT = NPAD // NS
GRP = 8


_sc_mesh = plsc.VectorSubcoreMesh(core_axis_name="c", subcore_axis_name="s")
_sc_params = pltpu.CompilerParams(needs_layout_passes=False)


def _deg_body(dst_flat, deg_out, idx_v, hist_v):
    c = lax.axis_index("c")
    s = lax.axis_index("s")
    wid = s * NC + c

    zeros16 = jnp.zeros((16,), jnp.float32)

    def zero_body(i, carry):
        hist_v[pl.ds(i * 16, 16)] = zeros16
        return carry

    lax.fori_loop(0, NPAD // 16, zero_body, 0)

    pltpu.sync_copy(dst_flat.at[pl.ds(wid * EPT, EPT)], idx_v)

    ones16 = jnp.ones((16,), jnp.float32)

    def acc_body(j, carry):
        dv = idx_v[pl.ds(j * 16, 16)]
        plsc.addupdate_scatter(hist_v, [dv], ones16)
        return carry

    lax.fori_loop(0, EPT // 16, acc_body, 0)

    pltpu.sync_copy(hist_v, deg_out.at[wid])


_deg_kernel = functools.partial(
    pl.kernel,
    out_type=jax.ShapeDtypeStruct((NW, NPAD), jnp.float32),
    mesh=_sc_mesh,
    compiler_params=_sc_params,
    scratch_types=[
        pltpu.VMEM((EPT,), jnp.int32),
        pltpu.VMEM((NPAD,), jnp.float32),
    ],
)(_deg_body)


def _scale_body(deg_ref, x_ref, y_ref, dinv_ref):
    degsum = jnp.sum(deg_ref[...], axis=0, keepdims=True) + 1.0
    dinv_row = lax.rsqrt(degsum)
    dinv_col = dinv_row.reshape(NPAD, 1)
    dinv_ref[...] = dinv_col
    y_ref[...] = x_ref[...] * dinv_col


def _scale_call(deg_p, x_ext):
    return pl.pallas_call(
        _scale_body,
        out_shape=(
            jax.ShapeDtypeStruct((NPAD, F), jnp.float32),
            jax.ShapeDtypeStruct((NPAD, 1), jnp.float32),
        ),
    )(deg_p, x_ext)


def _prop_body(src3, dst3, y_hbm, zeros_blk, zp_out,
               src_v, dst_v, rows0, rows1, z_sp, sem0, sem1):
    c = lax.axis_index("c")
    s = lax.axis_index("s")
    wid = s * NC + c

    pltpu.sync_copy(zeros_blk, rows0)
    for i in range(NPT // CH):
        pltpu.sync_copy(rows0, z_sp.at[pl.ds(s * NPT + i * CH, CH)])

    plsc.subcore_barrier()

    def group_body(g, carry):
        pltpu.sync_copy(src3.at[wid, pl.ds(g * GRP, GRP)], src_v)
        pltpu.sync_copy(dst3.at[wid, pl.ds(g * GRP, GRP)], dst_v)
        pltpu.async_copy(y_hbm.at[src_v.at[0]], rows0, sem0)

        def pair_body(t, carry2):
            j0 = t * 2
            j1 = j0 + 1
            pltpu.make_async_copy(y_hbm.at[src_v.at[j0]], rows0, sem0).wait()
            pltpu.async_copy(y_hbm.at[src_v.at[j1]], rows1, sem1)
            pltpu.sync_copy(rows0, z_sp.at[dst_v.at[j0]], add=True)
            pltpu.make_async_copy(y_hbm.at[src_v.at[j1]], rows1, sem1).wait()

            @pl.when(j1 + 1 < GRP)
            def _():
                pltpu.async_copy(y_hbm.at[src_v.at[j1 + 1]], rows0, sem0)

            pltpu.sync_copy(rows1, z_sp.at[dst_v.at[j1]], add=True)
            return carry2

        lax.fori_loop(0, GRP // 2, pair_body, 0)
        return carry

    lax.fori_loop(0, CHUNKS // GRP, group_body, 0)

    plsc.subcore_barrier()

    for i in range(NPT // CH):
        off = s * NPT + i * CH
        pltpu.sync_copy(z_sp.at[pl.ds(off, CH)], rows0)
        pltpu.sync_copy(rows0, zp_out.at[c, pl.ds(off, CH)])


_prop_kernel = functools.partial(
    pl.kernel,
    out_type=jax.ShapeDtypeStruct((NC, NPAD, F), jnp.float32),
    mesh=_sc_mesh,
    compiler_params=_sc_params,
    scratch_types=[
        pltpu.VMEM((GRP, CH), jnp.int32),
        pltpu.VMEM((GRP, CH), jnp.int32),
        pltpu.VMEM((CH, F), jnp.float32),
        pltpu.VMEM((CH, F), jnp.float32),
        pltpu.VMEM_SHARED((NPAD, F), jnp.float32),
        pltpu.SemaphoreType.DMA,
        pltpu.SemaphoreType.DMA,
    ],
)(_prop_body)


_DBLK = 2048


def _final_body(z_ref, y_ref, dinv_ref, x_ref, w_ref, b_ref, o_ref):
    z = z_ref[0] + z_ref[1] + y_ref[...]
    h = z * dinv_ref[...]
    out = lax.dot_general(h, w_ref[...], (((1,), (1,)), ((), ())),
                          preferred_element_type=jnp.float32)
    o_ref[...] = x_ref[...] + jnp.maximum(out + b_ref[...], 0.0)


def _final_call(z_p, y_ext, dinv_col, x_ext, W, b2):
    grid = NPAD // _DBLK
    return pl.pallas_call(
        _final_body,
        grid=(grid,),
        in_specs=[
            pl.BlockSpec((NC, _DBLK, F), lambda i: (0, i, 0)),
            pl.BlockSpec((_DBLK, F), lambda i: (i, 0)),
            pl.BlockSpec((_DBLK, 1), lambda i: (i, 0)),
            pl.BlockSpec((_DBLK, F), lambda i: (i, 0)),
            pl.BlockSpec((F, F), lambda i: (0, 0)),
            pl.BlockSpec((1, F), lambda i: (0, 0)),
        ],
        out_specs=pl.BlockSpec((_DBLK, F), lambda i: (i, 0)),
        out_shape=jax.ShapeDtypeStruct((N, F), jnp.float32),
    )(z_p, y_ext, dinv_col, x_ext, W, b2)


def kernel(x, edge_index, W, b):
    src = edge_index[0].astype(jnp.int32)
    dst = edge_index[1].astype(jnp.int32)
    pad = N + (jnp.arange(E_PAD - E, dtype=jnp.int32) % (NPAD - N))
    src3 = jnp.concatenate([src, pad]).reshape(NW, CHUNKS, CH)
    dst3 = jnp.concatenate([dst, pad]).reshape(NW, CHUNKS, CH)
    dst_flat = dst3.reshape(E_PAD)
    x_ext = jnp.concatenate(
        [x.astype(jnp.float32), jnp.zeros((NPAD - N, F), jnp.float32)])
    zeros_blk = jnp.zeros((CH, F), jnp.float32)
    b2 = b.reshape(1, F).astype(jnp.float32)

    deg_p = _deg_kernel(dst_flat)
    y_ext, dinv_col = _scale_call(deg_p, x_ext)
    z_p = _prop_kernel(src3, dst3, y_ext, zeros_blk)
    return _final_call(z_p, y_ext, dinv_col, x_ext, W.astype(jnp.float32), b2)
